# baseline (speedup 1.0000x reference)
import functools

import jax
import jax.numpy as jnp
from jax import lax
from jax.experimental import pallas as pl
from jax.experimental.pallas import tpu as pltpu
from jax.experimental.pallas import tpu_sc as plsc

_HIDDEN = 1024
_BATCH = 4
_SEQ = 2048
_TOK = _BATCH * _SEQ
_NW = 32
_SPW = _SEQ // _NW
_CH = 16
_NG = _SPW // _CH
_NCH = _BATCH * _NG
_NBUF = 5
_LANES = 16
_HG = _HIDDEN // _LANES

_mesh = plsc.VectorSubcoreMesh(core_axis_name="c", subcore_axis_name="s")


@functools.partial(
    pl.kernel,
    out_type=jax.ShapeDtypeStruct((_TOK, _HIDDEN), jnp.float32),
    mesh=_mesh,
    scratch_types=[
        pltpu.VMEM((_BATCH * _SPW,), jnp.int32),
        pltpu.VMEM((_NBUF, _CH, _HIDDEN), jnp.float32),
        pltpu.VMEM((2, _CH, _HIDDEN), jnp.float32),
        [pltpu.SemaphoreType.DMA] * _NBUF,
        [pltpu.SemaphoreType.DMA] * _NBUF,
        [pltpu.SemaphoreType.DMA] * 2,
    ],
)
def _emb_kernel(idx_hbm, wtab_hbm, ptab_hbm, out_hbm, idx_v, wbuf, pbuf,
                gsems, osems, psems):
    wid = lax.axis_index("s") * 2 + lax.axis_index("c")
    s0 = wid * _SPW

    for b in range(_BATCH):
        pltpu.sync_copy(idx_hbm.at[pl.ds(b * _SEQ + s0, _SPW)],
                        idx_v.at[pl.ds(b * _SPW, _SPW)])

    def gather(c):
        b, j = c % _BATCH, c // _BATCH
        buf = c % _NBUF
        return pltpu.async_copy(
            wtab_hbm.at[idx_v.at[pl.ds(b * _SPW + j * _CH, _CH)]],
            wbuf.at[buf], gsems[buf])

    def writeback(c):
        b, j = c % _BATCH, c // _BATCH
        buf = c % _NBUF
        row0 = b * _SEQ + s0 + j * _CH
        return pltpu.async_copy(wbuf.at[buf], out_hbm.at[pl.ds(row0, _CH)],
                                osems[buf])

    def load_pos(j):
        return pltpu.async_copy(ptab_hbm.at[pl.ds(s0 + j * _CH, _CH)],
                                pbuf.at[j % 2], psems[j % 2])

    p_descs = [None] * _NG
    p_descs[0] = load_pos(0)
    g_descs = [None] * _NCH
    o_descs = [None] * _NCH
    for c in range(_NBUF - 1):
        g_descs[c] = gather(c)

    for c in range(_NCH):
        buf = c % _NBUF
        j = c // _BATCH
        g_descs[c].wait()
        nxt = c + _NBUF - 1
        if nxt < _NCH:
            if c >= 1:
                o_descs[c - 1].wait()
            g_descs[nxt] = gather(nxt)
        if c % _BATCH == 0:
            p_descs[j].wait()
            if j + 1 < _NG:
                p_descs[j + 1] = load_pos(j + 1)
        pb = j % 2

        def row(r, _, buf=buf, pb=pb):
            for g in range(_HG):
                sl = pl.ds(g * _LANES, _LANES)
                plsc.addupdate(wbuf.at[buf, r, sl], pbuf[pb, r, sl])
            return 0

        lax.fori_loop(0, _CH, row, 0)
        o_descs[c] = writeback(c)

    for c in range(_NCH - _NBUF, _NCH):
        o_descs[c].wait()


def kernel(inputs, word_embeddings, position_embeddings):
    flat_idx = inputs.reshape(_TOK).astype(jnp.int32)
    out = _emb_kernel(flat_idx, word_embeddings, position_embeddings)
    return out.reshape(_BATCH, _SEQ, _HIDDEN)


# speedup vs baseline: 1.0534x; 1.0534x over previous
import functools

import jax
import jax.numpy as jnp
from jax import lax
from jax.experimental import pallas as pl
from jax.experimental.pallas import tpu as pltpu
from jax.experimental.pallas import tpu_sc as plsc

_HIDDEN = 1024
_BATCH = 4
_SEQ = 2048
_TOK = _BATCH * _SEQ
_NW = 32
_SPW = _SEQ // _NW
_CH = 16
_CPB = _SPW // _CH
_NCH = _BATCH * _CPB
_NBUF = 3
_LANES = 16
_HG = _HIDDEN // _LANES

_mesh = plsc.VectorSubcoreMesh(core_axis_name="c", subcore_axis_name="s")


@functools.partial(
    pl.kernel,
    out_type=jax.ShapeDtypeStruct((_TOK, _HIDDEN), jnp.float32),
    mesh=_mesh,
    scratch_types=[
        pltpu.VMEM((_BATCH * _SPW,), jnp.int32),
        pltpu.VMEM((_SPW, _HIDDEN), jnp.float32),
        [pltpu.VMEM((_CH, _HIDDEN), jnp.float32)] * _NBUF,
        [pltpu.SemaphoreType.DMA] * _NBUF,
        [pltpu.SemaphoreType.DMA] * _NBUF,
        pltpu.SemaphoreType.DMA,
    ],
)
def _emb_kernel(idx_hbm, wtab_hbm, ptab_hbm, out_hbm, idx_v, pos_v,
                wbufs, gsems, osems, psem):
    wid = lax.axis_index("s") * 2 + lax.axis_index("c")
    s0 = wid * _SPW

    pos_desc = pltpu.async_copy(ptab_hbm.at[pl.ds(s0, _SPW)], pos_v, psem)
    for b in range(_BATCH):
        pltpu.sync_copy(idx_hbm.at[pl.ds(b * _SEQ + s0, _SPW)],
                        idx_v.at[pl.ds(b * _SPW, _SPW)])

    def gather(c):
        buf = c % _NBUF
        return pltpu.async_copy(
            wtab_hbm.at[idx_v.at[pl.ds(c * _CH, _CH)]], wbufs[buf],
            gsems[buf])

    def writeback(c):
        b, j = divmod(c, _CPB)
        buf = c % _NBUF
        row0 = b * _SEQ + s0 + j * _CH
        return pltpu.async_copy(wbufs[buf], out_hbm.at[pl.ds(row0, _CH)],
                                osems[buf])

    g_descs = [None] * _NCH
    o_descs = [None] * _NCH
    for c in range(_NBUF - 1):
        g_descs[c] = gather(c)
    pos_desc.wait()

    for c in range(_NCH):
        buf = c % _NBUF
        g_descs[c].wait()
        nxt = c + _NBUF - 1
        if nxt < _NCH:
            if c >= 1:
                o_descs[c - 1].wait()
            g_descs[nxt] = gather(nxt)
        p0 = (c % _CPB) * _CH
        wb = wbufs[buf]

        def row(r, _, wb=wb, p0=p0):
            for g in range(_HG):
                sl = pl.ds(g * _LANES, _LANES)
                plsc.addupdate(wb.at[r, sl], pos_v[p0 + r, sl])
            return 0

        lax.fori_loop(0, _CH, row, 0)
        o_descs[c] = writeback(c)

    for c in range(_NCH - _NBUF, _NCH):
        o_descs[c].wait()


def kernel(inputs, word_embeddings, position_embeddings):
    flat_idx = inputs.reshape(_TOK).astype(jnp.int32)
    out = _emb_kernel(flat_idx, word_embeddings, position_embeddings)
    return out.reshape(_BATCH, _SEQ, _HIDDEN)


# speedup vs baseline: 1.2815x; 1.2165x over previous
import functools

import jax
import jax.numpy as jnp
from jax import lax
from jax.experimental import pallas as pl
from jax.experimental.pallas import tpu as pltpu
from jax.experimental.pallas import tpu_sc as plsc

_HIDDEN = 1024
_BATCH = 4
_SEQ = 2048
_TOK = _BATCH * _SEQ
_NW = 32
_SPW = _SEQ // _NW
_CH = 8
_NG = _SPW // _CH
_LANES = 16
_HG = _HIDDEN // _LANES

_mesh = plsc.VectorSubcoreMesh(core_axis_name="c", subcore_axis_name="s")


@functools.partial(
    pl.kernel,
    out_type=jax.ShapeDtypeStruct((_TOK, _HIDDEN), jnp.float32),
    mesh=_mesh,
    scratch_types=[
        pltpu.VMEM((_BATCH * _SPW,), jnp.int32),
        [pltpu.VMEM((_CH, _HIDDEN), jnp.float32)] * (2 * _BATCH),
        [pltpu.VMEM((_CH, _HIDDEN), jnp.float32)] * 2,
        [pltpu.SemaphoreType.DMA] * (2 * _BATCH),
        [pltpu.SemaphoreType.DMA] * (2 * _BATCH),
        [pltpu.SemaphoreType.DMA] * 2,
    ],
)
def _emb_kernel(idx_hbm, wtab_hbm, ptab_hbm, out_hbm, idx_v, wbufs, pbufs,
                gsems, osems, psems):
    wid = lax.axis_index("s") * 2 + lax.axis_index("c")
    s0 = wid * _SPW

    for b in range(_BATCH):
        pltpu.sync_copy(idx_hbm.at[pl.ds(b * _SEQ + s0, _SPW)],
                        idx_v.at[pl.ds(b * _SPW, _SPW)])

    def gather(j, b):
        slot = (j % 2) * _BATCH + b
        return pltpu.async_copy(
            wtab_hbm.at[idx_v.at[pl.ds(b * _SPW + j * _CH, _CH)]],
            wbufs[slot], gsems[slot])

    def writeback(j, b):
        slot = (j % 2) * _BATCH + b
        row0 = b * _SEQ + s0 + j * _CH
        return pltpu.async_copy(wbufs[slot], out_hbm.at[pl.ds(row0, _CH)],
                                osems[slot])

    def load_pos(j):
        return pltpu.async_copy(ptab_hbm.at[pl.ds(s0 + j * _CH, _CH)],
                                pbufs[j % 2], psems[j % 2])

    p_descs = [None] * _NG
    g_descs = [[None] * _BATCH for _ in range(_NG)]
    o_descs = [[None] * _BATCH for _ in range(_NG)]
    p_descs[0] = load_pos(0)
    for b in range(_BATCH):
        g_descs[0][b] = gather(0, b)

    for j in range(_NG):
        half = j % 2
        if j + 1 < _NG:
            p_descs[j + 1] = load_pos(j + 1)
            if j >= 1:
                for b in range(_BATCH):
                    o_descs[j - 1][b].wait()
            for b in range(_BATCH):
                g_descs[j + 1][b] = gather(j + 1, b)
        p_descs[j].wait()
        for b in range(_BATCH):
            g_descs[j][b].wait()
        wset = wbufs[half * _BATCH:(half + 1) * _BATCH]
        pb = pbufs[half]

        def row(r, _, wset=wset, pb=pb):
            for g in range(_HG):
                sl = pl.ds(g * _LANES, _LANES)
                x = pb[r, sl]
                for wb in wset:
                    plsc.addupdate(wb.at[r, sl], x)
            return 0

        lax.fori_loop(0, _CH, row, 0)
        for b in range(_BATCH):
            o_descs[j][b] = writeback(j, b)

    for b in range(_BATCH):
        o_descs[_NG - 2][b].wait()
        o_descs[_NG - 1][b].wait()


def kernel(inputs, word_embeddings, position_embeddings):
    flat_idx = inputs.reshape(_TOK).astype(jnp.int32)
    out = _emb_kernel(flat_idx, word_embeddings, position_embeddings)
    return out.reshape(_BATCH, _SEQ, _HIDDEN)
